# baseline (speedup 1.0000x reference)
import functools

import jax
import jax.numpy as jnp
from jax import lax
from jax.experimental import pallas as pl
from jax.experimental.pallas import tpu as pltpu
from jax.experimental.pallas import tpu_sc as plsc

M = 100000
D = 64
B = 16384
NC, NS = 2, 16
NW = NC * NS
BPW = B // NW
BPT = B // NS
NBLK = 8
BLK = 12504
SCRAP = 512
CHUNK = 784
NCH = BPT // 128

_mesh = plsc.VectorSubcoreMesh(core_axis_name="c", subcore_axis_name="s")
_params = pltpu.CompilerParams(needs_layout_passes=False,
                               use_tc_tiling_on_sc=False)


def _bcast15(v):
    return lax.gather(
        v, jnp.full((16, 1), 15, jnp.int32),
        lax.GatherDimensionNumbers(offset_dims=(), collapsed_slice_dims=(0,),
                                   start_index_map=(0,)),
        slice_sizes=(1,), mode=lax.GatherScatterMode.PROMISE_IN_BOUNDS)


def _load_idx_2d(idx_hbm, base, n, idx1_v, idx2_v):
    pltpu.sync_copy(idx_hbm.at[pl.ds(base, n)], idx1_v)
    for i in range(n // 16):
        idx2_v[i // 8, pl.ds((i % 8) * 16, 16)] = idx1_v[pl.ds(i * 16, 16)]


@functools.partial(
    pl.kernel, mesh=_mesh,
    out_type=jax.ShapeDtypeStruct((B, D), jnp.float32),
    compiler_params=_params,
    scratch_types=[
        pltpu.VMEM((BPW,), jnp.int32),
        pltpu.VMEM((BPW // 128, 128), jnp.int32),
        pltpu.VMEM((BPW, D), jnp.float32),
    ],
)
def _sc_gather(mem_hbm, idx_hbm, old_hbm, idx1_v, idx2_v, rows_v):
    wid = lax.axis_index("s") * NC + lax.axis_index("c")
    _load_idx_2d(idx_hbm, wid * BPW, BPW, idx1_v, idx2_v)
    for j in range(BPW // 128):
        pltpu.sync_copy(mem_hbm.at[idx2_v.at[j]],
                        rows_v.at[pl.ds(j * 128, 128)])
    pltpu.sync_copy(rows_v, old_hbm.at[pl.ds(wid * BPW, BPW)])


@functools.partial(
    pl.kernel, mesh=_mesh,
    out_type=jax.ShapeDtypeStruct((M, D), jnp.float32),
    compiler_params=_params,
    scratch_types=[
        pltpu.VMEM_SHARED((BLK + SCRAP, D), jnp.float32),
        pltpu.VMEM_SHARED((BLK + SCRAP, D), jnp.float32),
        pltpu.VMEM((BPT,), jnp.int32),
        pltpu.VMEM((NCH, 128), jnp.int32),
        pltpu.VMEM((NCH, 128), jnp.int32),
        pltpu.VMEM((NCH, 128), jnp.int32),
        pltpu.VMEM((128, D), jnp.float32),
        pltpu.SemaphoreType.DMA,
        pltpu.SemaphoreType.DMA,
        pltpu.SemaphoreType.DMA,
    ],
)
def _sc_scatter(mem_hbm, delta_hbm, idx_hbm, out_hbm,
                accA_sh, accB_sh, idx1_v, idx_v, selj_v, sell_v, buf_v,
                semA, semB, semC):
    cid = lax.axis_index("c")
    sid = lax.axis_index("s")
    _load_idx_2d(idx_hbm, sid * BPT, BPT, idx1_v, idx_v)
    cs = jnp.minimum(sid * CHUNK, BLK - CHUNK)
    outcopy = [None, None]
    for b in range(NBLK // NC):
        acc_sh = accA_sh if b % 2 == 0 else accB_sh
        sem = semA if b % 2 == 0 else semB
        blk = cid * (NBLK // NC) + b
        start = jnp.minimum(blk * BLK, M - BLK)
        if outcopy[b % 2] is not None:
            outcopy[b % 2].wait()
        incopy = pltpu.async_copy(mem_hbm.at[pl.ds(start + cs, CHUNK)],
                                  acc_sh.at[pl.ds(cs, CHUNK)], semC)
        for i in range(BPT // 16):
            lane = lax.iota(jnp.int32, 16) + i * 16
            sl = pl.ds((i % 8) * 16, 16)
            selj_v[i // 8, sl] = (lane + sid * 64) & (B - 1)
            sell_v[i // 8, sl] = BLK + ((lane + sid * 32) & (SCRAP - 1))
        off = jnp.zeros((16,), jnp.int32)
        for i in range(BPT // 16):
            iv = idx_v[i // 8, pl.ds((i % 8) * 16, 16)]
            inb = (iv >= start) & (iv < start + BLK)
            inc = plsc.cumsum(jnp.where(inb, 1, 0))
            pos = jnp.maximum(off + inc - 1, 0)
            row = lax.shift_right_logical(pos, 7)
            col = pos & 127
            jrow = lax.iota(jnp.int32, 16) + (sid * BPT + i * 16)
            plsc.store_scatter(selj_v, [row, col], jrow, mask=inb)
            plsc.store_scatter(sell_v, [row, col], iv - start, mask=inb)
            off = off + _bcast15(inc)
        incopy.wait()
        plsc.subcore_barrier()
        for c in range(NCH):
            @pl.when(jnp.any(off > c * 128))
            def _():
                pltpu.sync_copy(delta_hbm.at[selj_v.at[c]], buf_v)
                pltpu.sync_copy(buf_v, acc_sh.at[sell_v.at[c]], add=True)
        plsc.subcore_barrier()
        outcopy[b % 2] = pltpu.async_copy(
            acc_sh.at[pl.ds(cs, CHUNK)],
            out_hbm.at[pl.ds(start + cs, CHUNK)], sem)
    for h in outcopy:
        if h is not None:
            h.wait()


def _delta_body(old_ref, val_ref, wg_ref, bg_ref, wv_ref, out_ref):
    old = old_ref[...]
    v = val_ref[...]
    pre = jnp.dot(old + v, wg_ref[...], preferred_element_type=jnp.float32)
    gate = jax.nn.sigmoid(pre + bg_ref[...])
    upd = jnp.tanh(jnp.dot(v, wv_ref[...], preferred_element_type=jnp.float32))
    out_ref[...] = gate * (upd - old)


_TCB = 4096
_BP = B // 2


def _tc_delta(old_p, val_p, W2g, b2, W2v):
    return pl.pallas_call(
        _delta_body,
        grid=(_BP // _TCB,),
        in_specs=[
            pl.BlockSpec((_TCB, 2 * D), lambda i: (i, 0)),
            pl.BlockSpec((_TCB, 2 * D), lambda i: (i, 0)),
            pl.BlockSpec((2 * D, 2 * D), lambda i: (0, 0)),
            pl.BlockSpec((1, 2 * D), lambda i: (0, 0)),
            pl.BlockSpec((2 * D, 2 * D), lambda i: (0, 0)),
        ],
        out_specs=pl.BlockSpec((_TCB, 2 * D), lambda i: (i, 0)),
        out_shape=jax.ShapeDtypeStruct((_BP, 2 * D), jnp.float32),
    )(old_p, val_p, W2g, b2, W2v)


def _blockdiag2(w):
    z = jnp.zeros((D, D), w.dtype)
    return jnp.concatenate(
        [jnp.concatenate([w, z], 1), jnp.concatenate([z, w], 1)], 0)


def kernel(memory, indices, values, W_gate, b_gate, W_val):
    idx = indices.astype(jnp.int32)
    old = _sc_gather(memory, idx)
    delta_p = _tc_delta(old.reshape(_BP, 2 * D), values.reshape(_BP, 2 * D),
                        _blockdiag2(W_gate), jnp.tile(b_gate, 2).reshape(1, 2 * D),
                        _blockdiag2(W_val))
    return _sc_scatter(memory, delta_p.reshape(B, D), idx)
